# baseline (speedup 1.0000x reference)
import jax
import jax.numpy as jnp
from jax import lax
from jax.experimental import pallas as pl
from jax.experimental.pallas import tpu as pltpu
from jax.experimental.pallas import tpu_sc as plsc

N_NODES = 10000
N_EDGES = 320000
D = 128
NC = 2
NS = 16
NW = NC * NS
CHUNK = 128
CPW = 80
EPW = CPW * CHUNK
EP = EPW * NW
NP = 10112
RPT = NP // NS
RB = 1264
NBUF = 2

_MESH = plsc.VectorSubcoreMesh(core_axis_name="c", subcore_axis_name="s")


def _sc_hist_body(dst_hbm, zeros_hbm, ones_hbm, hist_out,
                  hist_sh, ones_v, didx_v, sem):
    c = lax.axis_index("c")
    s = lax.axis_index("s")
    wid = s * NC + c
    row0 = pl.multiple_of(s * RPT, 8)
    pltpu.sync_copy(zeros_hbm.at[pl.ds(row0, RPT)], hist_sh.at[pl.ds(row0, RPT)])
    pltpu.sync_copy(ones_hbm, ones_v)
    pltpu.sync_copy(dst_hbm.at[wid], didx_v)
    plsc.subcore_barrier()

    def fire(j, carry):
        pltpu.async_copy(ones_v, hist_sh.at[didx_v.at[j]], sem, add=True)
        return carry

    lax.fori_loop(0, CPW, fire, 0)

    def drain(j, carry):
        pltpu.make_async_copy(ones_v, hist_sh.at[didx_v.at[0]], sem).wait()
        return carry

    lax.fori_loop(0, CPW, drain, 0)
    plsc.subcore_barrier()
    pltpu.sync_copy(hist_sh.at[pl.ds(row0, RPT)], hist_out.at[c, pl.ds(row0, RPT)])


_sc_hist = pl.kernel(
    _sc_hist_body,
    out_type=jax.ShapeDtypeStruct((NC, NP, D), jnp.float32),
    mesh=_MESH,
    scratch_types=[
        pltpu.VMEM_SHARED((NP, D), jnp.float32),
        pltpu.VMEM((CHUNK, D), jnp.float32),
        pltpu.VMEM((CPW, CHUNK), jnp.int32),
        pltpu.SemaphoreType.DMA,
    ],
)


def _sc_scatter_body(hs_hbm, src_hbm, dst_hbm, zeros_hbm, part_out,
                     acc_sh, rows_v, sidx_v, didx_v,
                     gsem0, gsem1, dsem0, dsem1):
    gsems = (gsem0, gsem1)
    dsems = (dsem0, dsem1)
    c = lax.axis_index("c")
    s = lax.axis_index("s")
    wid = s * NC + c
    row0 = pl.multiple_of(s * RPT, 8)
    pltpu.sync_copy(src_hbm.at[wid], sidx_v)
    pltpu.sync_copy(zeros_hbm.at[pl.ds(row0, RPT)], acc_sh.at[pl.ds(row0, RPT)])
    plsc.subcore_barrier()

    for b in range(NBUF):
        pltpu.async_copy(hs_hbm.at[sidx_v.at[b]], rows_v.at[b], gsems[b])
        pltpu.async_copy(dst_hbm.at[wid, b], didx_v.at[b], dsems[b])

    def body(i, carry):
        for b in range(NBUF):
            j = i * NBUF + b
            pltpu.make_async_copy(hs_hbm.at[sidx_v.at[j]], rows_v.at[b],
                                  gsems[b]).wait()
            pltpu.make_async_copy(dst_hbm.at[wid, j], didx_v.at[b],
                                  dsems[b]).wait()
            pltpu.sync_copy(rows_v.at[b], acc_sh.at[didx_v.at[b]], add=True)

            @pl.when(j + NBUF < CPW)
            def _():
                pltpu.async_copy(hs_hbm.at[sidx_v.at[j + NBUF]], rows_v.at[b],
                                 gsems[b])
                pltpu.async_copy(dst_hbm.at[wid, j + NBUF], didx_v.at[b],
                                 dsems[b])
        return carry

    lax.fori_loop(0, CPW // NBUF, body, 0)
    plsc.subcore_barrier()
    pltpu.sync_copy(acc_sh.at[pl.ds(row0, RPT)], part_out.at[c, pl.ds(row0, RPT)])


_sc_scatter = pl.kernel(
    _sc_scatter_body,
    out_type=jax.ShapeDtypeStruct((NC, NP, D), jnp.float32),
    mesh=_MESH,
    scratch_types=[
        pltpu.VMEM_SHARED((NP, D), jnp.float32),
        pltpu.VMEM((NBUF, CHUNK, D), jnp.float32),
        pltpu.VMEM((CPW, CHUNK), jnp.int32),
        pltpu.VMEM((NBUF, CHUNK), jnp.int32),
        pltpu.SemaphoreType.DMA,
        pltpu.SemaphoreType.DMA,
        pltpu.SemaphoreType.DMA,
        pltpu.SemaphoreType.DMA,
    ],
)


def _dis(hist_ref):
    deg = hist_ref[0, :, 0:1] + hist_ref[1, :, 0:1] + 1.0
    return lax.rsqrt(deg)


def _tc_first_body(x_ref, w_ref, hist_ref, hs_ref):
    dis = _dis(hist_ref)
    hs_ref[...] = dis * jnp.dot(x_ref[...], w_ref[...],
                                preferred_element_type=jnp.float32)


_tc_first = pl.pallas_call(
    _tc_first_body,
    grid=(NP // RB,),
    in_specs=[
        pl.BlockSpec((RB, D), lambda i: (i, 0)),
        pl.BlockSpec((D, D), lambda i: (0, 0)),
        pl.BlockSpec((NC, RB, D), lambda i: (0, i, 0)),
    ],
    out_specs=pl.BlockSpec((RB, D), lambda i: (i, 0)),
    out_shape=jax.ShapeDtypeStruct((NP, D), jnp.float32),
)


def _tc_mid_body(hist_ref, p_ref, hs_ref, w_ref, b_ref, out_ref):
    dis = _dis(hist_ref)
    acc = p_ref[0] + p_ref[1] + hs_ref[...]
    h2 = jnp.maximum(dis * acc + b_ref[...], 0.0)
    out_ref[...] = dis * jnp.dot(h2, w_ref[...],
                                 preferred_element_type=jnp.float32)


_tc_mid = pl.pallas_call(
    _tc_mid_body,
    grid=(NP // RB,),
    in_specs=[
        pl.BlockSpec((NC, RB, D), lambda i: (0, i, 0)),
        pl.BlockSpec((NC, RB, D), lambda i: (0, i, 0)),
        pl.BlockSpec((RB, D), lambda i: (i, 0)),
        pl.BlockSpec((D, D), lambda i: (0, 0)),
        pl.BlockSpec((1, D), lambda i: (0, 0)),
    ],
    out_specs=pl.BlockSpec((RB, D), lambda i: (i, 0)),
    out_shape=jax.ShapeDtypeStruct((NP, D), jnp.float32),
)


def _tc_last_body(hist_ref, p_ref, hs_ref, b_ref, out_ref):
    dis = _dis(hist_ref)
    acc = p_ref[0] + p_ref[1] + hs_ref[...]
    out_ref[...] = dis * acc + b_ref[...]


_tc_last = pl.pallas_call(
    _tc_last_body,
    grid=(NP // RB,),
    in_specs=[
        pl.BlockSpec((NC, RB, D), lambda i: (0, i, 0)),
        pl.BlockSpec((NC, RB, D), lambda i: (0, i, 0)),
        pl.BlockSpec((RB, D), lambda i: (i, 0)),
        pl.BlockSpec((1, D), lambda i: (0, 0)),
    ],
    out_specs=pl.BlockSpec((RB, D), lambda i: (i, 0)),
    out_shape=jax.ShapeDtypeStruct((NP, D), jnp.float32),
)


def kernel(x, edge_index, W1, b1, W2, b2):
    pad_e = jnp.full((EP - N_EDGES,), N_NODES, jnp.int32)
    srcp = jnp.concatenate([edge_index[0], pad_e]).reshape(NW, CPW, CHUNK)
    dstp = jnp.concatenate([edge_index[1], pad_e]).reshape(NW, CPW, CHUNK)
    xp = jnp.pad(x, ((0, NP - N_NODES), (0, 0)))
    zeros_nd = jnp.zeros((NP, D), jnp.float32)
    ones_ch = jnp.ones((CHUNK, D), jnp.float32)

    hist = _sc_hist(dstp, zeros_nd, ones_ch)
    hs1 = _tc_first(xp, W1, hist)
    p = _sc_scatter(hs1, srcp, dstp, zeros_nd)
    hs2 = _tc_mid(hist, p, hs1, W2, b1.reshape(1, D))
    q = _sc_scatter(hs2, srcp, dstp, zeros_nd)
    out = _tc_last(hist, q, hs2, b2.reshape(1, D))
    return out[:N_NODES]


# speedup vs baseline: 1.1028x; 1.1028x over previous
import jax
import jax.numpy as jnp
from jax import lax
from jax.experimental import pallas as pl
from jax.experimental.pallas import tpu as pltpu
from jax.experimental.pallas import tpu_sc as plsc

N_NODES = 10000
N_EDGES = 320000
D = 128
NC = 2
NS = 16
NW = NC * NS
CHUNK = 128
CPW = 80
TOTCH = CPW * NW
EP = TOTCH * CHUNK
K0 = 128
K1 = 2 * CPW - K0
NP = 10112
RPT = NP // NS
RB = 1264
NBUF = 2

_MESH = plsc.VectorSubcoreMesh(core_axis_name="c", subcore_axis_name="s")


def _sc_hist_body(dst_hbm, zeros_hbm, ones_hbm, hist_out,
                  hist_sh, ones_v, didx_v, sem):
    c = lax.axis_index("c")
    s = lax.axis_index("s")
    wid = s * NC + c
    row0 = pl.multiple_of(s * RPT, 8)
    pltpu.sync_copy(zeros_hbm.at[pl.ds(row0, RPT)], hist_sh.at[pl.ds(row0, RPT)])
    pltpu.sync_copy(ones_hbm, ones_v)
    cb = pl.multiple_of(wid * CPW, 8)
    pltpu.sync_copy(dst_hbm.at[pl.ds(cb, CPW)], didx_v)
    plsc.subcore_barrier()

    def fire(j, carry):
        pltpu.async_copy(ones_v, hist_sh.at[didx_v.at[j]], sem, add=True)
        return carry

    lax.fori_loop(0, CPW, fire, 0)

    def drain(j, carry):
        pltpu.make_async_copy(ones_v, hist_sh.at[didx_v.at[0]], sem).wait()
        return carry

    lax.fori_loop(0, CPW, drain, 0)
    plsc.subcore_barrier()
    pltpu.sync_copy(hist_sh.at[pl.ds(row0, RPT)], hist_out.at[c, pl.ds(row0, RPT)])


_sc_hist = pl.kernel(
    _sc_hist_body,
    out_type=jax.ShapeDtypeStruct((NC, NP, D), jnp.float32),
    mesh=_MESH,
    scratch_types=[
        pltpu.VMEM_SHARED((NP, D), jnp.float32),
        pltpu.VMEM((CHUNK, D), jnp.float32),
        pltpu.VMEM((CPW, CHUNK), jnp.int32),
        pltpu.SemaphoreType.DMA,
    ],
)


def _edge_loop(hs_hbm, src_hbm, dst_hbm, acc_sh, rows_v, sidx_v, didx_v,
               gsems, dsems, cb, K):
    pltpu.sync_copy(src_hbm.at[pl.ds(cb, K)], sidx_v.at[pl.ds(0, K)])
    for b in range(NBUF):
        pltpu.async_copy(hs_hbm.at[sidx_v.at[b]], rows_v.at[b], gsems[b])
        pltpu.async_copy(dst_hbm.at[cb + b], didx_v.at[b], dsems[b])

    def body(i, carry):
        for b in range(NBUF):
            j = i * NBUF + b
            pltpu.make_async_copy(hs_hbm.at[sidx_v.at[j]], rows_v.at[b],
                                  gsems[b]).wait()
            pltpu.make_async_copy(dst_hbm.at[cb + j], didx_v.at[b],
                                  dsems[b]).wait()
            pltpu.sync_copy(rows_v.at[b], acc_sh.at[didx_v.at[b]], add=True)

            @pl.when(j + NBUF < K)
            def _():
                pltpu.async_copy(hs_hbm.at[sidx_v.at[j + NBUF]], rows_v.at[b],
                                 gsems[b])
                pltpu.async_copy(dst_hbm.at[cb + j + NBUF], didx_v.at[b],
                                 dsems[b])
        return carry

    lax.fori_loop(0, K // NBUF, body, 0)


def _sc_scatter_body(hs_hbm, src_hbm, dst_hbm, zeros_hbm, part_out,
                     acc_sh, rows_v, sidx_v, didx_v,
                     gsem0, gsem1, dsem0, dsem1):
    gsems = (gsem0, gsem1)
    dsems = (dsem0, dsem1)
    c = lax.axis_index("c")
    s = lax.axis_index("s")
    row0 = pl.multiple_of(s * RPT, 8)
    pltpu.sync_copy(zeros_hbm.at[pl.ds(row0, RPT)], acc_sh.at[pl.ds(row0, RPT)])
    plsc.subcore_barrier()

    @pl.when(c == 0)
    def _():
        cb = pl.multiple_of(s * K0, 8)
        _edge_loop(hs_hbm, src_hbm, dst_hbm, acc_sh, rows_v, sidx_v, didx_v,
                   gsems, dsems, cb, K0)

    @pl.when(c == 1)
    def _():
        cb = pl.multiple_of(NS * K0 + s * K1, 8)
        _edge_loop(hs_hbm, src_hbm, dst_hbm, acc_sh, rows_v, sidx_v, didx_v,
                   gsems, dsems, cb, K1)

    plsc.subcore_barrier()
    pltpu.sync_copy(acc_sh.at[pl.ds(row0, RPT)], part_out.at[c, pl.ds(row0, RPT)])


_sc_scatter = pl.kernel(
    _sc_scatter_body,
    out_type=jax.ShapeDtypeStruct((NC, NP, D), jnp.float32),
    mesh=_MESH,
    scratch_types=[
        pltpu.VMEM_SHARED((NP, D), jnp.float32),
        pltpu.VMEM((NBUF, CHUNK, D), jnp.float32),
        pltpu.VMEM((max(K0, K1), CHUNK), jnp.int32),
        pltpu.VMEM((NBUF, CHUNK), jnp.int32),
        pltpu.SemaphoreType.DMA,
        pltpu.SemaphoreType.DMA,
        pltpu.SemaphoreType.DMA,
        pltpu.SemaphoreType.DMA,
    ],
)


def _dis(hist_ref):
    deg = hist_ref[0, :, 0:1] + hist_ref[1, :, 0:1] + 1.0
    return lax.rsqrt(deg)


def _tc_first_body(x_ref, w_ref, hist_ref, hs_ref):
    dis = _dis(hist_ref)
    hs_ref[...] = dis * jnp.dot(x_ref[...], w_ref[...],
                                preferred_element_type=jnp.float32)


_tc_first = pl.pallas_call(
    _tc_first_body,
    grid=(NP // RB,),
    in_specs=[
        pl.BlockSpec((RB, D), lambda i: (i, 0)),
        pl.BlockSpec((D, D), lambda i: (0, 0)),
        pl.BlockSpec((NC, RB, D), lambda i: (0, i, 0)),
    ],
    out_specs=pl.BlockSpec((RB, D), lambda i: (i, 0)),
    out_shape=jax.ShapeDtypeStruct((NP, D), jnp.float32),
)


def _tc_mid_body(hist_ref, p_ref, hs_ref, w_ref, b_ref, out_ref):
    dis = _dis(hist_ref)
    acc = p_ref[0] + p_ref[1] + hs_ref[...]
    h2 = jnp.maximum(dis * acc + b_ref[...], 0.0)
    out_ref[...] = dis * jnp.dot(h2, w_ref[...],
                                 preferred_element_type=jnp.float32)


_tc_mid = pl.pallas_call(
    _tc_mid_body,
    grid=(NP // RB,),
    in_specs=[
        pl.BlockSpec((NC, RB, D), lambda i: (0, i, 0)),
        pl.BlockSpec((NC, RB, D), lambda i: (0, i, 0)),
        pl.BlockSpec((RB, D), lambda i: (i, 0)),
        pl.BlockSpec((D, D), lambda i: (0, 0)),
        pl.BlockSpec((1, D), lambda i: (0, 0)),
    ],
    out_specs=pl.BlockSpec((RB, D), lambda i: (i, 0)),
    out_shape=jax.ShapeDtypeStruct((NP, D), jnp.float32),
)


def _tc_last_body(hist_ref, p_ref, hs_ref, b_ref, out_ref):
    dis = _dis(hist_ref)
    acc = p_ref[0] + p_ref[1] + hs_ref[...]
    out_ref[...] = dis * acc + b_ref[...]


_tc_last = pl.pallas_call(
    _tc_last_body,
    grid=(NP // RB,),
    in_specs=[
        pl.BlockSpec((NC, RB, D), lambda i: (0, i, 0)),
        pl.BlockSpec((NC, RB, D), lambda i: (0, i, 0)),
        pl.BlockSpec((RB, D), lambda i: (i, 0)),
        pl.BlockSpec((1, D), lambda i: (0, 0)),
    ],
    out_specs=pl.BlockSpec((RB, D), lambda i: (i, 0)),
    out_shape=jax.ShapeDtypeStruct((NP, D), jnp.float32),
)


def kernel(x, edge_index, W1, b1, W2, b2):
    pad_e = jnp.full((EP - N_EDGES,), N_NODES, jnp.int32)
    srcp = jnp.concatenate([edge_index[0], pad_e]).reshape(TOTCH, CHUNK)
    dstp = jnp.concatenate([edge_index[1], pad_e]).reshape(TOTCH, CHUNK)
    xp = jnp.pad(x, ((0, NP - N_NODES), (0, 0)))
    zeros_nd = jnp.zeros((NP, D), jnp.float32)
    ones_ch = jnp.ones((CHUNK, D), jnp.float32)

    hist = _sc_hist(dstp, zeros_nd, ones_ch)
    hs1 = _tc_first(xp, W1, hist)
    p = _sc_scatter(hs1, srcp, dstp, zeros_nd)
    hs2 = _tc_mid(hist, p, hs1, W2, b1.reshape(1, D))
    q = _sc_scatter(hs2, srcp, dstp, zeros_nd)
    out = _tc_last(hist, q, hs2, b2.reshape(1, D))
    return out[:N_NODES]


# speedup vs baseline: 1.1037x; 1.0008x over previous
import jax
import jax.numpy as jnp
from jax import lax
from jax.experimental import pallas as pl
from jax.experimental.pallas import tpu as pltpu
from jax.experimental.pallas import tpu_sc as plsc

N_NODES = 10000
N_EDGES = 320000
D = 128
NC = 2
NS = 16
NW = NC * NS
CHUNK = 128
CPW = 80
TOTCH = CPW * NW
EP = TOTCH * CHUNK
K0 = 128
K1 = 2 * CPW - K0
NP = 10112
RPT = NP // NS
RB = 1264
NBUF = 2

_MESH = plsc.VectorSubcoreMesh(core_axis_name="c", subcore_axis_name="s")


def _sc_hist_body(dst_hbm, zeros_hbm, ones_hbm, hist_out,
                  hist_sh, ones_v, didx_v, sem):
    c = lax.axis_index("c")
    s = lax.axis_index("s")
    wid = s * NC + c
    row0 = pl.multiple_of(s * RPT, 8)
    pltpu.sync_copy(zeros_hbm.at[pl.ds(row0, RPT)], hist_sh.at[pl.ds(row0, RPT)])
    pltpu.sync_copy(ones_hbm, ones_v)
    cb = pl.multiple_of(wid * CPW, 8)
    pltpu.sync_copy(dst_hbm.at[pl.ds(cb, CPW)], didx_v)
    plsc.subcore_barrier()

    def fire(j, carry):
        pltpu.async_copy(ones_v, hist_sh.at[didx_v.at[j]], sem, add=True)
        return carry

    lax.fori_loop(0, CPW, fire, 0)

    def drain(j, carry):
        pltpu.make_async_copy(ones_v, hist_sh.at[didx_v.at[0]], sem).wait()
        return carry

    lax.fori_loop(0, CPW, drain, 0)
    plsc.subcore_barrier()
    pltpu.sync_copy(hist_sh.at[pl.ds(row0, RPT)], hist_out.at[c, pl.ds(row0, RPT)])


_sc_hist = pl.kernel(
    _sc_hist_body,
    out_type=jax.ShapeDtypeStruct((NC, NP, D), jnp.float32),
    mesh=_MESH,
    scratch_types=[
        pltpu.VMEM_SHARED((NP, D), jnp.float32),
        pltpu.VMEM((CHUNK, D), jnp.float32),
        pltpu.VMEM((CPW, CHUNK), jnp.int32),
        pltpu.SemaphoreType.DMA,
    ],
)


def _edge_loop(hs_hbm, src_hbm, dst_hbm, acc_sh, rows_v, sidx_v, didx_v,
               gsems, dsems, cb, K):
    pltpu.sync_copy(src_hbm.at[pl.ds(cb, K)], sidx_v.at[pl.ds(0, K)])
    for b in range(NBUF):
        pltpu.async_copy(hs_hbm.at[sidx_v.at[b]], rows_v.at[b], gsems[b])
        pltpu.async_copy(dst_hbm.at[cb + b], didx_v.at[b], dsems[b])

    def body(i, carry):
        for b in range(NBUF):
            j = i * NBUF + b
            pltpu.make_async_copy(hs_hbm.at[sidx_v.at[j]], rows_v.at[b],
                                  gsems[b]).wait()
            pltpu.make_async_copy(dst_hbm.at[cb + j], didx_v.at[b],
                                  dsems[b]).wait()
            pltpu.sync_copy(rows_v.at[b], acc_sh.at[didx_v.at[b]], add=True)

            @pl.when(j + NBUF < K)
            def _():
                pltpu.async_copy(hs_hbm.at[sidx_v.at[j + NBUF]], rows_v.at[b],
                                 gsems[b])
                pltpu.async_copy(dst_hbm.at[cb + j + NBUF], didx_v.at[b],
                                 dsems[b])
        return carry

    lax.fori_loop(0, K // NBUF, body, 0)


def _sc_scatter_body(hs_hbm, src_hbm, dst_hbm, zeros_hbm, part_out,
                     acc_sh, rows_v, sidx_v, didx_v,
                     gsem0, gsem1, dsem0, dsem1):
    gsems = (gsem0, gsem1)
    dsems = (dsem0, dsem1)
    c = lax.axis_index("c")
    s = lax.axis_index("s")
    row0 = pl.multiple_of(s * RPT, 8)
    with jax.named_scope("acc_zero"):
        pltpu.sync_copy(zeros_hbm.at[pl.ds(row0, RPT)], acc_sh.at[pl.ds(row0, RPT)])
        plsc.subcore_barrier()

    with jax.named_scope("edges"):
        @pl.when(c == 0)
        def _():
            cb = pl.multiple_of(s * K0, 8)
            _edge_loop(hs_hbm, src_hbm, dst_hbm, acc_sh, rows_v, sidx_v, didx_v,
                       gsems, dsems, cb, K0)

        @pl.when(c == 1)
        def _():
            cb = pl.multiple_of(NS * K0 + s * K1, 8)
            _edge_loop(hs_hbm, src_hbm, dst_hbm, acc_sh, rows_v, sidx_v, didx_v,
                       gsems, dsems, cb, K1)

    with jax.named_scope("drain"):
        plsc.subcore_barrier()
        pltpu.sync_copy(acc_sh.at[pl.ds(row0, RPT)], part_out.at[c, pl.ds(row0, RPT)])


_sc_scatter = pl.kernel(
    _sc_scatter_body,
    out_type=jax.ShapeDtypeStruct((NC, NP, D), jnp.float32),
    mesh=_MESH,
    scratch_types=[
        pltpu.VMEM_SHARED((NP, D), jnp.float32),
        pltpu.VMEM((NBUF, CHUNK, D), jnp.float32),
        pltpu.VMEM((max(K0, K1), CHUNK), jnp.int32),
        pltpu.VMEM((NBUF, CHUNK), jnp.int32),
        pltpu.SemaphoreType.DMA,
        pltpu.SemaphoreType.DMA,
        pltpu.SemaphoreType.DMA,
        pltpu.SemaphoreType.DMA,
    ],
)


def _dis(hist_ref):
    deg = hist_ref[0, :, 0:1] + hist_ref[1, :, 0:1] + 1.0
    return lax.rsqrt(deg)


def _tc_first_body(x_ref, w_ref, hist_ref, hs_ref):
    dis = _dis(hist_ref)
    hs_ref[...] = dis * jnp.dot(x_ref[...], w_ref[...],
                                preferred_element_type=jnp.float32)


_tc_first = pl.pallas_call(
    _tc_first_body,
    grid=(NP // RB,),
    in_specs=[
        pl.BlockSpec((RB, D), lambda i: (i, 0)),
        pl.BlockSpec((D, D), lambda i: (0, 0)),
        pl.BlockSpec((NC, RB, D), lambda i: (0, i, 0)),
    ],
    out_specs=pl.BlockSpec((RB, D), lambda i: (i, 0)),
    out_shape=jax.ShapeDtypeStruct((NP, D), jnp.float32),
)


def _tc_mid_body(hist_ref, p_ref, hs_ref, w_ref, b_ref, out_ref):
    dis = _dis(hist_ref)
    acc = p_ref[0] + p_ref[1] + hs_ref[...]
    h2 = jnp.maximum(dis * acc + b_ref[...], 0.0)
    out_ref[...] = dis * jnp.dot(h2, w_ref[...],
                                 preferred_element_type=jnp.float32)


_tc_mid = pl.pallas_call(
    _tc_mid_body,
    grid=(NP // RB,),
    in_specs=[
        pl.BlockSpec((NC, RB, D), lambda i: (0, i, 0)),
        pl.BlockSpec((NC, RB, D), lambda i: (0, i, 0)),
        pl.BlockSpec((RB, D), lambda i: (i, 0)),
        pl.BlockSpec((D, D), lambda i: (0, 0)),
        pl.BlockSpec((1, D), lambda i: (0, 0)),
    ],
    out_specs=pl.BlockSpec((RB, D), lambda i: (i, 0)),
    out_shape=jax.ShapeDtypeStruct((NP, D), jnp.float32),
)


def _tc_last_body(hist_ref, p_ref, hs_ref, b_ref, out_ref):
    dis = _dis(hist_ref)
    acc = p_ref[0] + p_ref[1] + hs_ref[...]
    out_ref[...] = dis * acc + b_ref[...]


_tc_last = pl.pallas_call(
    _tc_last_body,
    grid=(NP // RB,),
    in_specs=[
        pl.BlockSpec((NC, RB, D), lambda i: (0, i, 0)),
        pl.BlockSpec((NC, RB, D), lambda i: (0, i, 0)),
        pl.BlockSpec((RB, D), lambda i: (i, 0)),
        pl.BlockSpec((1, D), lambda i: (0, 0)),
    ],
    out_specs=pl.BlockSpec((RB, D), lambda i: (i, 0)),
    out_shape=jax.ShapeDtypeStruct((NP, D), jnp.float32),
)


def kernel(x, edge_index, W1, b1, W2, b2):
    pad_e = jnp.full((EP - N_EDGES,), N_NODES, jnp.int32)
    srcp = jnp.concatenate([edge_index[0], pad_e]).reshape(TOTCH, CHUNK)
    dstp = jnp.concatenate([edge_index[1], pad_e]).reshape(TOTCH, CHUNK)
    xp = jnp.pad(x, ((0, NP - N_NODES), (0, 0)))
    zeros_nd = jnp.zeros((NP, D), jnp.float32)
    ones_ch = jnp.ones((CHUNK, D), jnp.float32)

    hist = _sc_hist(dstp, zeros_nd, ones_ch)
    hs1 = _tc_first(xp, W1, hist)
    p = _sc_scatter(hs1, srcp, dstp, zeros_nd)
    hs2 = _tc_mid(hist, p, hs1, W2, b1.reshape(1, D))
    q = _sc_scatter(hs2, srcp, dstp, zeros_nd)
    out = _tc_last(hist, q, hs2, b2.reshape(1, D))
    return out[:N_NODES]


# speedup vs baseline: 3.4306x; 3.1084x over previous
import jax
import jax.numpy as jnp
from jax import lax
from jax.experimental import pallas as pl
from jax.experimental.pallas import tpu as pltpu
from jax.experimental.pallas import tpu_sc as plsc

N_NODES = 10000
N_EDGES = 320000
D = 128
NC = 2
NS = 16
NW = NC * NS
CHUNK = 128
CPW = 80
TOTCH = CPW * NW
EP = TOTCH * CHUNK
K0 = 80
K1 = 2 * CPW - K0
NP = 10112
RPT = NP // NS
RB = 1264
NBUF = 2

_MESH = plsc.VectorSubcoreMesh(core_axis_name="c", subcore_axis_name="s")


def _sc_hist_body(dst_hbm, zeros_hbm, ones_hbm, hist_out,
                  hist_sh, ones_v, didx_v, sem):
    c = lax.axis_index("c")
    s = lax.axis_index("s")
    wid = s * NC + c
    row0 = pl.multiple_of(s * RPT, 8)
    pltpu.sync_copy(zeros_hbm.at[pl.ds(row0, RPT)], hist_sh.at[pl.ds(row0, RPT)])
    pltpu.sync_copy(ones_hbm, ones_v)
    cb = pl.multiple_of(wid * CPW, 8)
    pltpu.sync_copy(dst_hbm.at[pl.ds(cb, CPW)], didx_v)
    plsc.subcore_barrier()

    def fire(j, carry):
        pltpu.async_copy(ones_v, hist_sh.at[didx_v.at[j]], sem, add=True)
        return carry

    lax.fori_loop(0, CPW, fire, 0)

    def drain(j, carry):
        pltpu.make_async_copy(ones_v, hist_sh.at[didx_v.at[0]], sem).wait()
        return carry

    lax.fori_loop(0, CPW, drain, 0)
    plsc.subcore_barrier()
    pltpu.sync_copy(hist_sh.at[pl.ds(row0, RPT)], hist_out.at[c, pl.ds(row0, RPT)])


_sc_hist = pl.kernel(
    _sc_hist_body,
    out_type=jax.ShapeDtypeStruct((NC, NP, D), jnp.float32),
    mesh=_MESH,
    scratch_types=[
        pltpu.VMEM_SHARED((NP, D), jnp.float32),
        pltpu.VMEM((CHUNK, D), jnp.float32),
        pltpu.VMEM((CPW, CHUNK), jnp.int32),
        pltpu.SemaphoreType.DMA,
    ],
)


def _edge_loop(hs_hbm, src_hbm, dst_hbm, acc_sh, rows_v, sidx_v, didx_v,
               gsems, dsems, cb, K):
    pltpu.sync_copy(src_hbm.at[pl.ds(cb, K)], sidx_v.at[pl.ds(0, K)])
    for b in range(NBUF):
        pltpu.async_copy(hs_hbm.at[sidx_v.at[b]], rows_v.at[b], gsems[b])
        pltpu.async_copy(dst_hbm.at[cb + b], didx_v.at[b], dsems[b])

    def body(i, carry):
        for b in range(NBUF):
            j = i * NBUF + b
            pltpu.make_async_copy(hs_hbm.at[sidx_v.at[j]], rows_v.at[b],
                                  gsems[b]).wait()
            pltpu.make_async_copy(dst_hbm.at[cb + j], didx_v.at[b],
                                  dsems[b]).wait()
            pltpu.sync_copy(rows_v.at[b], acc_sh.at[didx_v.at[b]], add=True)

            @pl.when(j + NBUF < K)
            def _():
                pltpu.async_copy(hs_hbm.at[sidx_v.at[j + NBUF]], rows_v.at[b],
                                 gsems[b])
                pltpu.async_copy(dst_hbm.at[cb + j + NBUF], didx_v.at[b],
                                 dsems[b])
        return carry

    lax.fori_loop(0, K // NBUF, body, 0)


def _sc_scatter_body(hs_hbm, src_hbm, dst_hbm, zeros_hbm, part_out,
                     acc_sh, rows_v, sidx_v, didx_v,
                     gsem0, gsem1, dsem0, dsem1):
    gsems = (gsem0, gsem1)
    dsems = (dsem0, dsem1)
    c = lax.axis_index("c")
    s = lax.axis_index("s")
    row0 = pl.multiple_of(s * RPT, 8)
    with jax.named_scope("acc_zero"):
        pltpu.sync_copy(zeros_hbm.at[pl.ds(row0, RPT)], acc_sh.at[pl.ds(row0, RPT)])
        plsc.subcore_barrier()

    with jax.named_scope("edges"):
        @pl.when(c == 0)
        def _():
            cb = pl.multiple_of(s * K0, 8)
            _edge_loop(hs_hbm, src_hbm, dst_hbm, acc_sh, rows_v, sidx_v, didx_v,
                       gsems, dsems, cb, K0)

        @pl.when(c == 1)
        def _():
            cb = pl.multiple_of(NS * K0 + s * K1, 8)
            _edge_loop(hs_hbm, src_hbm, dst_hbm, acc_sh, rows_v, sidx_v, didx_v,
                       gsems, dsems, cb, K1)

    with jax.named_scope("drain"):
        plsc.subcore_barrier()
        pltpu.sync_copy(acc_sh.at[pl.ds(row0, RPT)], part_out.at[c, pl.ds(row0, RPT)])


_sc_scatter = pl.kernel(
    _sc_scatter_body,
    out_type=jax.ShapeDtypeStruct((NC, NP, D), jnp.float32),
    mesh=_MESH,
    scratch_types=[
        pltpu.VMEM_SHARED((NP, D), jnp.float32),
        pltpu.VMEM((NBUF, CHUNK, D), jnp.float32),
        pltpu.VMEM((max(K0, K1), CHUNK), jnp.int32),
        pltpu.VMEM((NBUF, CHUNK), jnp.int32),
        pltpu.SemaphoreType.DMA,
        pltpu.SemaphoreType.DMA,
        pltpu.SemaphoreType.DMA,
        pltpu.SemaphoreType.DMA,
    ],
)


def _dis(hist_ref):
    deg = hist_ref[0, :, 0:1] + hist_ref[1, :, 0:1] + 1.0
    return lax.rsqrt(deg)


def _tc_first_body(x_ref, w_ref, hist_ref, hs_ref):
    dis = _dis(hist_ref)
    hs_ref[...] = dis * jnp.dot(x_ref[...], w_ref[...],
                                preferred_element_type=jnp.float32)


_tc_first = pl.pallas_call(
    _tc_first_body,
    grid=(NP // RB,),
    in_specs=[
        pl.BlockSpec((RB, D), lambda i: (i, 0)),
        pl.BlockSpec((D, D), lambda i: (0, 0)),
        pl.BlockSpec((NC, RB, D), lambda i: (0, i, 0)),
    ],
    out_specs=pl.BlockSpec((RB, D), lambda i: (i, 0)),
    out_shape=jax.ShapeDtypeStruct((NP, D), jnp.float32),
)


def _tc_mid_body(hist_ref, p_ref, hs_ref, w_ref, b_ref, out_ref):
    dis = _dis(hist_ref)
    acc = p_ref[0] + p_ref[1] + hs_ref[...]
    h2 = jnp.maximum(dis * acc + b_ref[...], 0.0)
    out_ref[...] = dis * jnp.dot(h2, w_ref[...],
                                 preferred_element_type=jnp.float32)


_tc_mid = pl.pallas_call(
    _tc_mid_body,
    grid=(NP // RB,),
    in_specs=[
        pl.BlockSpec((NC, RB, D), lambda i: (0, i, 0)),
        pl.BlockSpec((NC, RB, D), lambda i: (0, i, 0)),
        pl.BlockSpec((RB, D), lambda i: (i, 0)),
        pl.BlockSpec((D, D), lambda i: (0, 0)),
        pl.BlockSpec((1, D), lambda i: (0, 0)),
    ],
    out_specs=pl.BlockSpec((RB, D), lambda i: (i, 0)),
    out_shape=jax.ShapeDtypeStruct((NP, D), jnp.float32),
)


def _tc_last_body(hist_ref, p_ref, hs_ref, b_ref, out_ref):
    dis = _dis(hist_ref)
    acc = p_ref[0] + p_ref[1] + hs_ref[...]
    out_ref[...] = dis * acc + b_ref[...]


_tc_last = pl.pallas_call(
    _tc_last_body,
    grid=(NP // RB,),
    in_specs=[
        pl.BlockSpec((NC, RB, D), lambda i: (0, i, 0)),
        pl.BlockSpec((NC, RB, D), lambda i: (0, i, 0)),
        pl.BlockSpec((RB, D), lambda i: (i, 0)),
        pl.BlockSpec((1, D), lambda i: (0, 0)),
    ],
    out_specs=pl.BlockSpec((RB, D), lambda i: (i, 0)),
    out_shape=jax.ShapeDtypeStruct((NP, D), jnp.float32),
)


def kernel(x, edge_index, W1, b1, W2, b2):
    pad_e = N_NODES + (jnp.arange(EP - N_EDGES, dtype=jnp.int32)
                       % (NP - N_NODES))
    srcp = jnp.concatenate([edge_index[0], pad_e]).reshape(TOTCH, CHUNK)
    dstp = jnp.concatenate([edge_index[1], pad_e]).reshape(TOTCH, CHUNK)
    xp = jnp.pad(x, ((0, NP - N_NODES), (0, 0)))
    zeros_nd = jnp.zeros((NP, D), jnp.float32)
    ones_ch = jnp.ones((CHUNK, D), jnp.float32)

    hist = _sc_hist(dstp, zeros_nd, ones_ch)
    hs1 = _tc_first(xp, W1, hist)
    p = _sc_scatter(hs1, srcp, dstp, zeros_nd)
    hs2 = _tc_mid(hist, p, hs1, W2, b1.reshape(1, D))
    q = _sc_scatter(hs2, srcp, dstp, zeros_nd)
    out = _tc_last(hist, q, hs2, b2.reshape(1, D))
    return out[:N_NODES]


# speedup vs baseline: 3.4567x; 1.0076x over previous
import jax
import jax.numpy as jnp
from jax import lax
from jax.experimental import pallas as pl
from jax.experimental.pallas import tpu as pltpu
from jax.experimental.pallas import tpu_sc as plsc

N_NODES = 10000
N_EDGES = 320000
D = 128
NC = 2
NS = 16
NW = NC * NS
CHUNK = 128
CPW = 80
TOTCH = CPW * NW
EP = TOTCH * CHUNK
K0 = 80
K1 = 2 * CPW - K0
NP = 10112
RPT = NP // NS
RB = 1000
HW = 128
NBUF = 2

_MESH = plsc.VectorSubcoreMesh(core_axis_name="c", subcore_axis_name="s")


def _sc_hist_body(dst_hbm, zeros_hbm, ones_hbm, hist_out,
                  hist_sh, ones_v, didx_v, sem):
    c = lax.axis_index("c")
    s = lax.axis_index("s")
    wid = s * NC + c
    row0 = pl.multiple_of(s * RPT, 8)
    pltpu.sync_copy(zeros_hbm.at[pl.ds(row0, RPT)], hist_sh.at[pl.ds(row0, RPT)])
    pltpu.sync_copy(ones_hbm, ones_v)
    cb = pl.multiple_of(wid * CPW, 8)
    pltpu.sync_copy(dst_hbm.at[pl.ds(cb, CPW)], didx_v)
    plsc.subcore_barrier()

    def fire(j, carry):
        pltpu.async_copy(ones_v, hist_sh.at[didx_v.at[j]], sem, add=True)
        return carry

    lax.fori_loop(0, CPW, fire, 0)

    def drain(j, carry):
        pltpu.make_async_copy(ones_v, hist_sh.at[didx_v.at[0]], sem).wait()
        return carry

    lax.fori_loop(0, CPW, drain, 0)
    plsc.subcore_barrier()
    pltpu.sync_copy(hist_sh.at[pl.ds(row0, RPT)], hist_out.at[c, pl.ds(row0, RPT)])


_sc_hist = pl.kernel(
    _sc_hist_body,
    out_type=jax.ShapeDtypeStruct((NC, NP, HW), jnp.float32),
    mesh=_MESH,
    scratch_types=[
        pltpu.VMEM_SHARED((NP, HW), jnp.float32),
        pltpu.VMEM((CHUNK, HW), jnp.float32),
        pltpu.VMEM((CPW, CHUNK), jnp.int32),
        pltpu.SemaphoreType.DMA,
    ],
)


def _edge_loop(hs_hbm, src_hbm, dst_hbm, acc_sh, rows_v, sidx_v, didx_v,
               gsems, dsems, cb, K):
    pltpu.sync_copy(src_hbm.at[pl.ds(cb, K)], sidx_v.at[pl.ds(0, K)])
    for b in range(NBUF):
        pltpu.async_copy(hs_hbm.at[sidx_v.at[b]], rows_v.at[b], gsems[b])
        pltpu.async_copy(dst_hbm.at[cb + b], didx_v.at[b], dsems[b])

    def body(i, carry):
        for b in range(NBUF):
            j = i * NBUF + b
            pltpu.make_async_copy(hs_hbm.at[sidx_v.at[j]], rows_v.at[b],
                                  gsems[b]).wait()
            pltpu.make_async_copy(dst_hbm.at[cb + j], didx_v.at[b],
                                  dsems[b]).wait()
            pltpu.sync_copy(rows_v.at[b], acc_sh.at[didx_v.at[b]], add=True)

            @pl.when(j + NBUF < K)
            def _():
                pltpu.async_copy(hs_hbm.at[sidx_v.at[j + NBUF]], rows_v.at[b],
                                 gsems[b])
                pltpu.async_copy(dst_hbm.at[cb + j + NBUF], didx_v.at[b],
                                 dsems[b])
        return carry

    lax.fori_loop(0, K // NBUF, body, 0)


def _sc_scatter_body(hs_hbm, src_hbm, dst_hbm, zeros_hbm, part_out,
                     acc_sh, rows_v, sidx_v, didx_v,
                     gsem0, gsem1, dsem0, dsem1):
    gsems = (gsem0, gsem1)
    dsems = (dsem0, dsem1)
    c = lax.axis_index("c")
    s = lax.axis_index("s")
    row0 = pl.multiple_of(s * RPT, 8)
    with jax.named_scope("acc_zero"):
        pltpu.sync_copy(zeros_hbm.at[pl.ds(row0, RPT)], acc_sh.at[pl.ds(row0, RPT)])
        plsc.subcore_barrier()

    with jax.named_scope("edges"):
        @pl.when(c == 0)
        def _():
            cb = pl.multiple_of(s * K0, 8)
            _edge_loop(hs_hbm, src_hbm, dst_hbm, acc_sh, rows_v, sidx_v, didx_v,
                       gsems, dsems, cb, K0)

        @pl.when(c == 1)
        def _():
            cb = pl.multiple_of(NS * K0 + s * K1, 8)
            _edge_loop(hs_hbm, src_hbm, dst_hbm, acc_sh, rows_v, sidx_v, didx_v,
                       gsems, dsems, cb, K1)

    with jax.named_scope("drain"):
        plsc.subcore_barrier()
        pltpu.sync_copy(acc_sh.at[pl.ds(row0, RPT)], part_out.at[c, pl.ds(row0, RPT)])


_sc_scatter = pl.kernel(
    _sc_scatter_body,
    out_type=jax.ShapeDtypeStruct((NC, NP, D), jnp.float32),
    mesh=_MESH,
    scratch_types=[
        pltpu.VMEM_SHARED((NP, D), jnp.float32),
        pltpu.VMEM((NBUF, CHUNK, D), jnp.float32),
        pltpu.VMEM((max(K0, K1), CHUNK), jnp.int32),
        pltpu.VMEM((NBUF, CHUNK), jnp.int32),
        pltpu.SemaphoreType.DMA,
        pltpu.SemaphoreType.DMA,
        pltpu.SemaphoreType.DMA,
        pltpu.SemaphoreType.DMA,
    ],
)


def _dis(hist_ref):
    deg = hist_ref[0, :, 0:1] + hist_ref[1, :, 0:1] + 1.0
    return lax.rsqrt(deg)


def _tc_first_body(x_ref, w_ref, hist_ref, hs_ref):
    dis = _dis(hist_ref)
    hs_ref[...] = dis * jnp.dot(x_ref[...], w_ref[...],
                                preferred_element_type=jnp.float32)


_tc_first = pl.pallas_call(
    _tc_first_body,
    grid=(N_NODES // RB,),
    in_specs=[
        pl.BlockSpec((RB, D), lambda i: (i, 0)),
        pl.BlockSpec((D, D), lambda i: (0, 0)),
        pl.BlockSpec((NC, RB, HW), lambda i: (0, i, 0)),
    ],
    out_specs=pl.BlockSpec((RB, D), lambda i: (i, 0)),
    out_shape=jax.ShapeDtypeStruct((N_NODES, D), jnp.float32),
)


def _tc_mid_body(hist_ref, p_ref, hs_ref, w_ref, b_ref, out_ref):
    dis = _dis(hist_ref)
    acc = p_ref[0] + p_ref[1] + hs_ref[...]
    h2 = jnp.maximum(dis * acc + b_ref[...], 0.0)
    out_ref[...] = dis * jnp.dot(h2, w_ref[...],
                                 preferred_element_type=jnp.float32)


_tc_mid = pl.pallas_call(
    _tc_mid_body,
    grid=(N_NODES // RB,),
    in_specs=[
        pl.BlockSpec((NC, RB, HW), lambda i: (0, i, 0)),
        pl.BlockSpec((NC, RB, D), lambda i: (0, i, 0)),
        pl.BlockSpec((RB, D), lambda i: (i, 0)),
        pl.BlockSpec((D, D), lambda i: (0, 0)),
        pl.BlockSpec((1, D), lambda i: (0, 0)),
    ],
    out_specs=pl.BlockSpec((RB, D), lambda i: (i, 0)),
    out_shape=jax.ShapeDtypeStruct((N_NODES, D), jnp.float32),
)


def _tc_last_body(hist_ref, p_ref, hs_ref, b_ref, out_ref):
    dis = _dis(hist_ref)
    acc = p_ref[0] + p_ref[1] + hs_ref[...]
    out_ref[...] = dis * acc + b_ref[...]


_tc_last = pl.pallas_call(
    _tc_last_body,
    grid=(N_NODES // RB,),
    in_specs=[
        pl.BlockSpec((NC, RB, HW), lambda i: (0, i, 0)),
        pl.BlockSpec((NC, RB, D), lambda i: (0, i, 0)),
        pl.BlockSpec((RB, D), lambda i: (i, 0)),
        pl.BlockSpec((1, D), lambda i: (0, 0)),
    ],
    out_specs=pl.BlockSpec((RB, D), lambda i: (i, 0)),
    out_shape=jax.ShapeDtypeStruct((N_NODES, D), jnp.float32),
)


def kernel(x, edge_index, W1, b1, W2, b2):
    npad = EP - N_EDGES
    pad_dst = N_NODES + (jnp.arange(npad, dtype=jnp.int32) % (NP - N_NODES))
    pad_src = jnp.arange(npad, dtype=jnp.int32) % N_NODES
    srcp = jnp.concatenate([edge_index[0], pad_src]).reshape(TOTCH, CHUNK)
    dstp = jnp.concatenate([edge_index[1], pad_dst]).reshape(TOTCH, CHUNK)
    zeros_nd = jnp.zeros((NP, D), jnp.float32)
    ones_ch = jnp.ones((CHUNK, HW), jnp.float32)

    hist = _sc_hist(dstp, zeros_nd, ones_ch)
    hs1 = _tc_first(x, W1, hist)
    p = _sc_scatter(hs1, srcp, dstp, zeros_nd)
    hs2 = _tc_mid(hist, p, hs1, W2, b1.reshape(1, D))
    q = _sc_scatter(hs2, srcp, dstp, zeros_nd)
    return _tc_last(hist, q, hs2, b2.reshape(1, D))


# speedup vs baseline: 3.7401x; 1.0820x over previous
import jax
import jax.numpy as jnp
from jax import lax
from jax.experimental import pallas as pl
from jax.experimental.pallas import tpu as pltpu
from jax.experimental.pallas import tpu_sc as plsc

N_NODES = 10000
N_EDGES = 320000
D = 128
NC = 2
NS = 16
NW = NC * NS
CHUNK = 128
CG = 64
CPW = 80
TOTCH = CPW * NW
EP = TOTCH * CHUNK
K0 = 80
K1 = 2 * CPW - K0
NP = 10112
RPT = NP // NS
RB = 1000
HW = 128
NBUF = 4

_MESH = plsc.VectorSubcoreMesh(core_axis_name="c", subcore_axis_name="s")


def _sc_hist_body(dst_hbm, zeros_hbm, ones_hbm, hist_out,
                  hist_sh, ones_v, didx_v, sem):
    c = lax.axis_index("c")
    s = lax.axis_index("s")
    wid = s * NC + c
    row0 = pl.multiple_of(s * RPT, 8)
    pltpu.sync_copy(zeros_hbm.at[pl.ds(row0, RPT)], hist_sh.at[pl.ds(row0, RPT)])
    pltpu.sync_copy(ones_hbm, ones_v)
    cb = pl.multiple_of(wid * CPW, 8)
    pltpu.sync_copy(dst_hbm.at[pl.ds(cb, CPW)], didx_v)
    plsc.subcore_barrier()

    def fire(j, carry):
        pltpu.async_copy(ones_v, hist_sh.at[didx_v.at[j]], sem, add=True)
        return carry

    lax.fori_loop(0, CPW, fire, 0)

    def drain(j, carry):
        pltpu.make_async_copy(ones_v, hist_sh.at[didx_v.at[0]], sem).wait()
        return carry

    lax.fori_loop(0, CPW, drain, 0)
    plsc.subcore_barrier()
    pltpu.sync_copy(hist_sh.at[pl.ds(row0, RPT)], hist_out.at[c, pl.ds(row0, RPT)])


_sc_hist = pl.kernel(
    _sc_hist_body,
    out_type=jax.ShapeDtypeStruct((NC, NP, HW), jnp.float32),
    mesh=_MESH,
    scratch_types=[
        pltpu.VMEM_SHARED((NP, HW), jnp.float32),
        pltpu.VMEM((CHUNK, HW), jnp.float32),
        pltpu.VMEM((CPW, CHUNK), jnp.int32),
        pltpu.SemaphoreType.DMA,
    ],
)


def _edge_loop(hs_hbm, src_hbm, dst_hbm, acc_sh, rows_v, sidx_v, didx_v,
               gsems, dsems, cb, K):
    pltpu.sync_copy(src_hbm.at[pl.ds(cb, K)], sidx_v.at[pl.ds(0, K)])
    kg = 2 * K

    def sidx_slice(j):
        return sidx_v.at[j // 2, pl.ds(pl.multiple_of((j % 2) * CG, 8), CG)]

    def dsrc(j):
        return dst_hbm.at[cb + j // 2, pl.ds(pl.multiple_of((j % 2) * CG, 8), CG)]

    for b in range(NBUF):
        pltpu.async_copy(hs_hbm.at[sidx_slice(b)], rows_v.at[b], gsems[b])
        pltpu.async_copy(dsrc(b), didx_v.at[b], dsems[b])

    def body(i, carry):
        for b in range(NBUF):
            j = i * NBUF + b
            pltpu.make_async_copy(hs_hbm.at[sidx_slice(j)], rows_v.at[b],
                                  gsems[b]).wait()
            pltpu.make_async_copy(dsrc(j), didx_v.at[b], dsems[b]).wait()
            pltpu.sync_copy(rows_v.at[b], acc_sh.at[didx_v.at[b]], add=True)

            @pl.when(j + NBUF < kg)
            def _():
                pltpu.async_copy(hs_hbm.at[sidx_slice(j + NBUF)], rows_v.at[b],
                                 gsems[b])
                pltpu.async_copy(dsrc(j + NBUF), didx_v.at[b], dsems[b])
        return carry

    lax.fori_loop(0, kg // NBUF, body, 0)


def _sc_scatter_body(hs_hbm, src_hbm, dst_hbm, zeros_hbm, part_out,
                     acc_sh, rows_v, sidx_v, didx_v,
                     gsem0, gsem1, gsem2, gsem3, dsem0, dsem1, dsem2, dsem3):
    gsems = (gsem0, gsem1, gsem2, gsem3)
    dsems = (dsem0, dsem1, dsem2, dsem3)
    c = lax.axis_index("c")
    s = lax.axis_index("s")
    row0 = pl.multiple_of(s * RPT, 8)
    with jax.named_scope("acc_zero"):
        pltpu.sync_copy(zeros_hbm.at[pl.ds(row0, RPT)], acc_sh.at[pl.ds(row0, RPT)])
        plsc.subcore_barrier()

    with jax.named_scope("edges"):
        @pl.when(c == 0)
        def _():
            cb = pl.multiple_of(s * K0, 8)
            _edge_loop(hs_hbm, src_hbm, dst_hbm, acc_sh, rows_v, sidx_v, didx_v,
                       gsems, dsems, cb, K0)

        @pl.when(c == 1)
        def _():
            cb = pl.multiple_of(NS * K0 + s * K1, 8)
            _edge_loop(hs_hbm, src_hbm, dst_hbm, acc_sh, rows_v, sidx_v, didx_v,
                       gsems, dsems, cb, K1)

    with jax.named_scope("drain"):
        plsc.subcore_barrier()
        pltpu.sync_copy(acc_sh.at[pl.ds(row0, RPT)], part_out.at[c, pl.ds(row0, RPT)])


_sc_scatter = pl.kernel(
    _sc_scatter_body,
    out_type=jax.ShapeDtypeStruct((NC, NP, D), jnp.float32),
    mesh=_MESH,
    scratch_types=[
        pltpu.VMEM_SHARED((NP, D), jnp.float32),
        pltpu.VMEM((NBUF, CG, D), jnp.float32),
        pltpu.VMEM((max(K0, K1), CHUNK), jnp.int32),
        pltpu.VMEM((NBUF, CG), jnp.int32),
        pltpu.SemaphoreType.DMA,
        pltpu.SemaphoreType.DMA,
        pltpu.SemaphoreType.DMA,
        pltpu.SemaphoreType.DMA,
        pltpu.SemaphoreType.DMA,
        pltpu.SemaphoreType.DMA,
        pltpu.SemaphoreType.DMA,
        pltpu.SemaphoreType.DMA,
    ],
)


def _dis(hist_ref):
    deg = hist_ref[0, :, 0:1] + hist_ref[1, :, 0:1] + 1.0
    return lax.rsqrt(deg)


def _tc_first_body(x_ref, w_ref, hist_ref, hs_ref):
    dis = _dis(hist_ref)
    hs_ref[...] = dis * jnp.dot(x_ref[...], w_ref[...],
                                preferred_element_type=jnp.float32)


_tc_first = pl.pallas_call(
    _tc_first_body,
    grid=(N_NODES // RB,),
    in_specs=[
        pl.BlockSpec((RB, D), lambda i: (i, 0)),
        pl.BlockSpec((D, D), lambda i: (0, 0)),
        pl.BlockSpec((NC, RB, HW), lambda i: (0, i, 0)),
    ],
    out_specs=pl.BlockSpec((RB, D), lambda i: (i, 0)),
    out_shape=jax.ShapeDtypeStruct((N_NODES, D), jnp.float32),
)


def _tc_mid_body(hist_ref, p_ref, hs_ref, w_ref, b_ref, out_ref):
    dis = _dis(hist_ref)
    acc = p_ref[0] + p_ref[1] + hs_ref[...]
    h2 = jnp.maximum(dis * acc + b_ref[...], 0.0)
    out_ref[...] = dis * jnp.dot(h2, w_ref[...],
                                 preferred_element_type=jnp.float32)


_tc_mid = pl.pallas_call(
    _tc_mid_body,
    grid=(N_NODES // RB,),
    in_specs=[
        pl.BlockSpec((NC, RB, HW), lambda i: (0, i, 0)),
        pl.BlockSpec((NC, RB, D), lambda i: (0, i, 0)),
        pl.BlockSpec((RB, D), lambda i: (i, 0)),
        pl.BlockSpec((D, D), lambda i: (0, 0)),
        pl.BlockSpec((1, D), lambda i: (0, 0)),
    ],
    out_specs=pl.BlockSpec((RB, D), lambda i: (i, 0)),
    out_shape=jax.ShapeDtypeStruct((N_NODES, D), jnp.float32),
)


def _tc_last_body(hist_ref, p_ref, hs_ref, b_ref, out_ref):
    dis = _dis(hist_ref)
    acc = p_ref[0] + p_ref[1] + hs_ref[...]
    out_ref[...] = dis * acc + b_ref[...]


_tc_last = pl.pallas_call(
    _tc_last_body,
    grid=(N_NODES // RB,),
    in_specs=[
        pl.BlockSpec((NC, RB, HW), lambda i: (0, i, 0)),
        pl.BlockSpec((NC, RB, D), lambda i: (0, i, 0)),
        pl.BlockSpec((RB, D), lambda i: (i, 0)),
        pl.BlockSpec((1, D), lambda i: (0, 0)),
    ],
    out_specs=pl.BlockSpec((RB, D), lambda i: (i, 0)),
    out_shape=jax.ShapeDtypeStruct((N_NODES, D), jnp.float32),
)


def kernel(x, edge_index, W1, b1, W2, b2):
    npad = EP - N_EDGES
    pad_dst = N_NODES + (jnp.arange(npad, dtype=jnp.int32) % (NP - N_NODES))
    pad_src = jnp.arange(npad, dtype=jnp.int32) % N_NODES
    srcp = jnp.concatenate([edge_index[0], pad_src]).reshape(TOTCH, CHUNK)
    dstp = jnp.concatenate([edge_index[1], pad_dst]).reshape(TOTCH, CHUNK)
    zeros_nd = jnp.zeros((NP, D), jnp.float32)
    ones_ch = jnp.ones((CHUNK, HW), jnp.float32)

    hist = _sc_hist(dstp, zeros_nd, ones_ch)
    hs1 = _tc_first(x, W1, hist)
    p = _sc_scatter(hs1, srcp, dstp, zeros_nd)
    hs2 = _tc_mid(hist, p, hs1, W2, b1.reshape(1, D))
    q = _sc_scatter(hs2, srcp, dstp, zeros_nd)
    return _tc_last(hist, q, hs2, b2.reshape(1, D))


# speedup vs baseline: 3.7547x; 1.0039x over previous
import jax
import jax.numpy as jnp
import numpy as np
from jax import lax
from jax.experimental import pallas as pl
from jax.experimental.pallas import tpu as pltpu
from jax.experimental.pallas import tpu_sc as plsc

N_NODES = 10000
N_EDGES = 320000
D = 128
NC = 2
NS = 16
NW = NC * NS
CHUNK = 128
CG = 64
CPW = 80
TOTCH = CPW * NW
EP = TOTCH * CHUNK
K0 = 80
K1 = 2 * CPW - K0
NP = 10112
RPT = NP // NS
RB = 1000
HW = 128
NBUF = 4

_MESH = plsc.VectorSubcoreMesh(core_axis_name="c", subcore_axis_name="s")

_NPAD = EP - N_EDGES
_PAD_DST = np.int32(N_NODES) + (np.arange(_NPAD, dtype=np.int32) % (NP - N_NODES))
_PAD_SRC = np.arange(_NPAD, dtype=np.int32) % N_NODES


def _sc_hist_body(dst_hbm, zeros_hbm, ones_hbm, hist_out,
                  hist_sh, ones_v, didx_v, sem):
    c = lax.axis_index("c")
    s = lax.axis_index("s")
    wid = s * NC + c
    row0 = pl.multiple_of(s * RPT, 8)
    pltpu.sync_copy(zeros_hbm.at[pl.ds(row0, RPT)], hist_sh.at[pl.ds(row0, RPT)])
    pltpu.sync_copy(ones_hbm, ones_v)
    cb = pl.multiple_of(wid * CPW, 8)
    pltpu.sync_copy(dst_hbm.at[pl.ds(cb, CPW)], didx_v)
    plsc.subcore_barrier()

    def fire(j, carry):
        pltpu.async_copy(ones_v, hist_sh.at[didx_v.at[j]], sem, add=True)
        return carry

    lax.fori_loop(0, CPW, fire, 0)

    def drain(j, carry):
        pltpu.make_async_copy(ones_v, hist_sh.at[didx_v.at[0]], sem).wait()
        return carry

    lax.fori_loop(0, CPW, drain, 0)
    plsc.subcore_barrier()
    pltpu.sync_copy(hist_sh.at[pl.ds(row0, RPT)], hist_out.at[c, pl.ds(row0, RPT)])


_sc_hist = pl.kernel(
    _sc_hist_body,
    out_type=jax.ShapeDtypeStruct((NC, NP, HW), jnp.float32),
    mesh=_MESH,
    scratch_types=[
        pltpu.VMEM_SHARED((NP, HW), jnp.float32),
        pltpu.VMEM((CHUNK, HW), jnp.float32),
        pltpu.VMEM((CPW, CHUNK), jnp.int32),
        pltpu.SemaphoreType.DMA,
    ],
)


def _edge_loop(hs_hbm, src_hbm, dst_hbm, acc_sh, rows_v, sidx_v, didx_v,
               gsems, dsems, cb, K):
    pltpu.sync_copy(src_hbm.at[pl.ds(cb, K)], sidx_v.at[pl.ds(0, K)])
    kg = 2 * K

    def sidx_slice(j):
        return sidx_v.at[j // 2, pl.ds(pl.multiple_of((j % 2) * CG, 8), CG)]

    def dsrc(j):
        return dst_hbm.at[cb + j // 2, pl.ds(pl.multiple_of((j % 2) * CG, 8), CG)]

    for b in range(NBUF):
        pltpu.async_copy(hs_hbm.at[sidx_slice(b)], rows_v.at[b], gsems[b])
        pltpu.async_copy(dsrc(b), didx_v.at[b], dsems[b])

    def body(i, carry):
        for b in range(NBUF):
            j = i * NBUF + b
            pltpu.make_async_copy(hs_hbm.at[sidx_slice(j)], rows_v.at[b],
                                  gsems[b]).wait()
            pltpu.make_async_copy(dsrc(j), didx_v.at[b], dsems[b]).wait()
            pltpu.sync_copy(rows_v.at[b], acc_sh.at[didx_v.at[b]], add=True)

            @pl.when(j + NBUF < kg)
            def _():
                pltpu.async_copy(hs_hbm.at[sidx_slice(j + NBUF)], rows_v.at[b],
                                 gsems[b])
                pltpu.async_copy(dsrc(j + NBUF), didx_v.at[b], dsems[b])
        return carry

    lax.fori_loop(0, kg // NBUF, body, 0)


def _sc_scatter_body(hs_hbm, src_hbm, dst_hbm, zeros_hbm, part_out,
                     acc_sh, rows_v, sidx_v, didx_v,
                     gsem0, gsem1, gsem2, gsem3, dsem0, dsem1, dsem2, dsem3):
    gsems = (gsem0, gsem1, gsem2, gsem3)
    dsems = (dsem0, dsem1, dsem2, dsem3)
    c = lax.axis_index("c")
    s = lax.axis_index("s")
    row0 = pl.multiple_of(s * RPT, 8)
    with jax.named_scope("acc_init"):
        @pl.when((c == 0) & (s < NS - 1))
        def _():
            pltpu.sync_copy(hs_hbm.at[pl.ds(row0, RPT)],
                            acc_sh.at[pl.ds(row0, RPT)])

        @pl.when((c == 0) & (s == NS - 1))
        def _():
            nreal = N_NODES - (NS - 1) * RPT
            pltpu.sync_copy(hs_hbm.at[pl.ds(row0, nreal)],
                            acc_sh.at[pl.ds(row0, nreal)])
            pltpu.sync_copy(zeros_hbm.at[pl.ds(N_NODES, NP - N_NODES)],
                            acc_sh.at[pl.ds(N_NODES, NP - N_NODES)])

        @pl.when(c == 1)
        def _():
            pltpu.sync_copy(zeros_hbm.at[pl.ds(row0, RPT)],
                            acc_sh.at[pl.ds(row0, RPT)])

        plsc.subcore_barrier()

    with jax.named_scope("edges"):
        @pl.when(c == 0)
        def _():
            cb = pl.multiple_of(s * K0, 8)
            _edge_loop(hs_hbm, src_hbm, dst_hbm, acc_sh, rows_v, sidx_v, didx_v,
                       gsems, dsems, cb, K0)

        @pl.when(c == 1)
        def _():
            cb = pl.multiple_of(NS * K0 + s * K1, 8)
            _edge_loop(hs_hbm, src_hbm, dst_hbm, acc_sh, rows_v, sidx_v, didx_v,
                       gsems, dsems, cb, K1)

    with jax.named_scope("drain"):
        plsc.subcore_barrier()
        pltpu.sync_copy(acc_sh.at[pl.ds(row0, RPT)], part_out.at[c, pl.ds(row0, RPT)])


_sc_scatter = pl.kernel(
    _sc_scatter_body,
    out_type=jax.ShapeDtypeStruct((NC, NP, D), jnp.float32),
    mesh=_MESH,
    scratch_types=[
        pltpu.VMEM_SHARED((NP, D), jnp.float32),
        pltpu.VMEM((NBUF, CG, D), jnp.float32),
        pltpu.VMEM((max(K0, K1), CHUNK), jnp.int32),
        pltpu.VMEM((NBUF, CG), jnp.int32),
        pltpu.SemaphoreType.DMA,
        pltpu.SemaphoreType.DMA,
        pltpu.SemaphoreType.DMA,
        pltpu.SemaphoreType.DMA,
        pltpu.SemaphoreType.DMA,
        pltpu.SemaphoreType.DMA,
        pltpu.SemaphoreType.DMA,
        pltpu.SemaphoreType.DMA,
    ],
)


def _dis(hist_ref):
    deg = hist_ref[0, :, 0:1] + hist_ref[1, :, 0:1] + 1.0
    return lax.rsqrt(deg)


def _tc_first_body(x_ref, w_ref, hist_ref, hs_ref):
    dis = _dis(hist_ref)
    hs_ref[...] = dis * jnp.dot(x_ref[...], w_ref[...],
                                preferred_element_type=jnp.float32)


_tc_first = pl.pallas_call(
    _tc_first_body,
    grid=(N_NODES // RB,),
    in_specs=[
        pl.BlockSpec((RB, D), lambda i: (i, 0)),
        pl.BlockSpec((D, D), lambda i: (0, 0)),
        pl.BlockSpec((NC, RB, HW), lambda i: (0, i, 0)),
    ],
    out_specs=pl.BlockSpec((RB, D), lambda i: (i, 0)),
    out_shape=jax.ShapeDtypeStruct((N_NODES, D), jnp.float32),
)


def _tc_mid_body(hist_ref, p_ref, w_ref, b_ref, out_ref):
    dis = _dis(hist_ref)
    acc = p_ref[0] + p_ref[1]
    h2 = jnp.maximum(dis * acc + b_ref[...], 0.0)
    out_ref[...] = dis * jnp.dot(h2, w_ref[...],
                                 preferred_element_type=jnp.float32)


_tc_mid = pl.pallas_call(
    _tc_mid_body,
    grid=(N_NODES // RB,),
    in_specs=[
        pl.BlockSpec((NC, RB, HW), lambda i: (0, i, 0)),
        pl.BlockSpec((NC, RB, D), lambda i: (0, i, 0)),
        pl.BlockSpec((D, D), lambda i: (0, 0)),
        pl.BlockSpec((1, D), lambda i: (0, 0)),
    ],
    out_specs=pl.BlockSpec((RB, D), lambda i: (i, 0)),
    out_shape=jax.ShapeDtypeStruct((N_NODES, D), jnp.float32),
)


def _tc_last_body(hist_ref, p_ref, b_ref, out_ref):
    dis = _dis(hist_ref)
    acc = p_ref[0] + p_ref[1]
    out_ref[...] = dis * acc + b_ref[...]


_tc_last = pl.pallas_call(
    _tc_last_body,
    grid=(N_NODES // RB,),
    in_specs=[
        pl.BlockSpec((NC, RB, HW), lambda i: (0, i, 0)),
        pl.BlockSpec((NC, RB, D), lambda i: (0, i, 0)),
        pl.BlockSpec((1, D), lambda i: (0, 0)),
    ],
    out_specs=pl.BlockSpec((RB, D), lambda i: (i, 0)),
    out_shape=jax.ShapeDtypeStruct((N_NODES, D), jnp.float32),
)


def kernel(x, edge_index, W1, b1, W2, b2):
    srcp = jnp.concatenate([edge_index[0], _PAD_SRC]).reshape(TOTCH, CHUNK)
    dstp = jnp.concatenate([edge_index[1], _PAD_DST]).reshape(TOTCH, CHUNK)
    zeros_nd = jnp.zeros((NP, D), jnp.float32)
    ones_ch = jnp.ones((CHUNK, HW), jnp.float32)

    hist = _sc_hist(dstp, zeros_nd, ones_ch)
    hs1 = _tc_first(x, W1, hist)
    p = _sc_scatter(hs1, srcp, dstp, zeros_nd)
    hs2 = _tc_mid(hist, p, W2, b1.reshape(1, D))
    q = _sc_scatter(hs2, srcp, dstp, zeros_nd)
    return _tc_last(hist, q, b2.reshape(1, D))
